# baseline (speedup 1.0000x reference)
import functools

import jax
import jax.numpy as jnp
from jax import lax
from jax.experimental import pallas as pl
from jax.experimental.pallas import tpu as pltpu
from jax.experimental.pallas import tpu_sc as plsc

_BATCH = 16384
_HIST = 50
_D = 32
_NC = 2
_NS = 16
_NW = _NC * _NS
_BT = 128
_NBT = _BATCH // _BT
_BT_PER_W = _NBT // _NW
_HG = 10
_NHG = _HIST // _HG


@functools.partial(
    pl.kernel,
    out_type=jax.ShapeDtypeStruct((_HIST, _D // 8, _NBT, 8, _BT),
                                  jnp.float32),
    mesh=plsc.VectorSubcoreMesh(core_axis_name="c", subcore_axis_name="s"),
    scratch_types=[
        pltpu.VMEM((_HIST, _BT), jnp.int32),
        pltpu.VMEM((_HG, _BT, _D), jnp.float32),
        pltpu.VMEM((_HG, _D // 8, 8, _BT), jnp.float32),
        pltpu.SemaphoreType.DMA,
        pltpu.SemaphoreType.DMA,
    ],
    compiler_params=pltpu.CompilerParams(use_tc_tiling_on_sc=False, needs_layout_passes=False),
)
def _gather_kernel(idxt_hbm, table_hbm, out_hbm, idxt_v, rows_v,
                   tile_v, gsem, wsem):
    wid = lax.axis_index("s") * _NC + lax.axis_index("c")
    lane = lax.iota(jnp.int32, 16)
    lanes16 = [lane + 16 * k for k in range(_BT // 16)]

    wback = None
    for jb in range(_BT_PER_W):
        jbg = wid * _BT_PER_W + jb
        pltpu.sync_copy(idxt_hbm.at[:, pl.ds(jbg * _BT, _BT)], idxt_v)

        for hg in range(_NHG):
            h0 = hg * _HG
            gathers = []
            for hq in range(_HG):
                gathers.append(pltpu.async_copy(
                    table_hbm.at[idxt_v.at[h0 + hq, :]], rows_v.at[hq],
                    gsem))
            if wback is not None:
                wback.wait()
                wback = None
            for g in gathers:
                g.wait()

            @pl.loop(0, _HG)
            def _x(hq):
                row = jnp.broadcast_to(hq, (16,)).astype(jnp.int32)

                @pl.loop(0, _D)
                def _xd(d):
                    dim = jnp.broadcast_to(d, (16,)).astype(jnp.int32)
                    for k in range(_BT // 16):
                        tile_v[hq, d // 8, d % 8, pl.ds(16 * k, 16)] = (
                            plsc.load_gather(rows_v,
                                             [row, lanes16[k], dim]))

            wback = pltpu.async_copy(
                tile_v, out_hbm.at[pl.ds(h0, _HG), :, jbg, :, :], wsem)
    wback.wait()


def kernel(input, table):
    out5 = _gather_kernel(input.astype(jnp.int32).T, table)
    return out5.transpose(2, 4, 0, 1, 3).reshape(_BATCH, _HIST, _D)


# speedup vs baseline: 1.1628x; 1.1628x over previous
import functools

import jax
import jax.numpy as jnp
from jax import lax
from jax.experimental import pallas as pl
from jax.experimental.pallas import tpu as pltpu
from jax.experimental.pallas import tpu_sc as plsc

_BATCH = 16384
_HIST = 50
_D = 32
_NC = 2
_NS = 16
_NW = _NC * _NS
_BW = _BATCH // _NW


@functools.partial(
    pl.kernel,
    out_type=jax.ShapeDtypeStruct((_BATCH, _HIST, _D), jnp.float32),
    mesh=plsc.VectorSubcoreMesh(core_axis_name="c", subcore_axis_name="s"),
    scratch_types=[
        pltpu.VMEM((_HIST, _BW), jnp.int32),
        pltpu.VMEM((2, _BW, _D), jnp.float32),
        pltpu.SemaphoreType.DMA,
        pltpu.SemaphoreType.DMA,
        pltpu.SemaphoreType.DMA,
        pltpu.SemaphoreType.DMA,
    ],
    compiler_params=pltpu.CompilerParams(use_tc_tiling_on_sc=False),
)
def _gather_kernel(idxt_hbm, table_hbm, out_hbm, idxt_v, rows_v,
                   gsem0, gsem1, wsem0, wsem1):
    wid = lax.axis_index("s") * _NC + lax.axis_index("c")
    b0 = wid * _BW
    pltpu.sync_copy(idxt_hbm.at[:, pl.ds(b0, _BW)], idxt_v)

    def _gwait(h, p, sem):
        pltpu.make_async_copy(table_hbm.at[idxt_v.at[h, :]], rows_v.at[p],
                              sem).wait()

    def _wwait(p, sem):
        pltpu.make_async_copy(rows_v.at[p], out_hbm.at[pl.ds(b0, _BW), 0, :],
                              sem).wait()

    @pl.loop(0, _HIST // 2)
    def _pair(t):
        h0 = 2 * t

        @pl.when(t >= 1)
        def _():
            _wwait(0, wsem0)
        pltpu.async_copy(table_hbm.at[idxt_v.at[h0, :]], rows_v.at[0],
                         gsem0)

        @pl.when(t >= 1)
        def _():
            _wwait(1, wsem1)
        pltpu.async_copy(table_hbm.at[idxt_v.at[h0 + 1, :]], rows_v.at[1],
                         gsem1)

        _gwait(h0, 0, gsem0)
        pltpu.async_copy(rows_v.at[0], out_hbm.at[pl.ds(b0, _BW), h0, :],
                         wsem0)
        _gwait(h0 + 1, 1, gsem1)
        pltpu.async_copy(rows_v.at[1],
                         out_hbm.at[pl.ds(b0, _BW), h0 + 1, :], wsem1)

    _wwait(0, wsem0)
    _wwait(1, wsem1)


def kernel(input, table):
    return _gather_kernel(input.astype(jnp.int32).T, table)
